# Initial kernel scaffold; baseline (speedup 1.0000x reference)
import functools

import jax
import jax.numpy as jnp
from jax import lax
from jax.experimental import pallas as pl
from jax.experimental.pallas import tpu as pltpu
from jax.experimental.pallas import tpu_sc as plsc

N = 10000
M = 320000
D_IN = 128
D_OUT = 129
DP = 144
NC = 2
NS = 16
NW = NC * NS
EDGES_PER_TILE = M // NW
CHUNK = 80
N_CHUNKS = EDGES_PER_TILE // CHUNK
ROWS_PER_TILE = N // NS
ZBLK = 125

_MESH = plsc.VectorSubcoreMesh(core_axis_name="c", subcore_axis_name="s")


@functools.partial(
    pl.kernel,
    out_type=jax.ShapeDtypeStruct((NC * N, DP), jnp.float32),
    mesh=_MESH,
    scratch_types=[
        pltpu.VMEM((CHUNK,), jnp.int32),
        pltpu.VMEM((CHUNK,), jnp.int32),
        pltpu.VMEM((CHUNK, DP), jnp.float32),
        pltpu.VMEM_SHARED((N, DP), jnp.float32),
        pltpu.SemaphoreType.DMA,
    ],
)
def _sc_gather_scatter_add(node_hbm, src_hbm, dst_hbm, zeros_hbm, out_hbm,
                           src_v, dst_v, rows_v, acc, sem):
    c = lax.axis_index("c")
    s = lax.axis_index("s")
    wid = s * NC + c
    row0 = s * ROWS_PER_TILE

    @pl.when(c == 0)
    def _():
        pltpu.sync_copy(node_hbm.at[pl.ds(row0, ROWS_PER_TILE)],
                        acc.at[pl.ds(row0, ROWS_PER_TILE)])

    @pl.when(c != 0)
    def _():
        for j in range(ROWS_PER_TILE // ZBLK):
            pltpu.sync_copy(zeros_hbm, acc.at[pl.ds(row0 + j * ZBLK, ZBLK)])

    plsc.subcore_barrier()

    e0 = wid * EDGES_PER_TILE

    def body(i, carry):
        base = e0 + i * CHUNK
        pltpu.sync_copy(src_hbm.at[pl.ds(base, CHUNK)], src_v)
        pltpu.sync_copy(dst_hbm.at[pl.ds(base, CHUNK)], dst_v)
        pltpu.async_copy(node_hbm.at[src_v], rows_v, sem).wait()
        pltpu.sync_copy(rows_v, acc.at[dst_v], add=True)
        return carry

    lax.fori_loop(0, N_CHUNKS, body, 0)

    plsc.subcore_barrier()

    pltpu.sync_copy(acc.at[pl.ds(row0, ROWS_PER_TILE)],
                    out_hbm.at[pl.ds(c * N + row0, ROWS_PER_TILE)])


def _combine_body(a_ref, b_ref, o_ref):
    o_ref[...] = (a_ref[...] + b_ref[...])[:, :D_OUT]


_combine = pl.pallas_call(
    _combine_body,
    grid=(10,),
    in_specs=[
        pl.BlockSpec((N // 10, DP), lambda i: (i, 0)),
        pl.BlockSpec((N // 10, DP), lambda i: (i + 10, 0)),
    ],
    out_specs=pl.BlockSpec((N // 10, D_OUT), lambda i: (i, 0)),
    out_shape=jax.ShapeDtypeStruct((N, D_OUT), jnp.float32),
)


def kernel(node, edge_index, eps_k):
    del eps_k
    rkey = jax.random.fold_in(jax.random.key(0), 42)
    rand = jax.random.uniform(
        rkey, (N, 1), minval=0.0, maxval=100.0, dtype=jnp.float32) / 100.0
    node_pad = jnp.concatenate(
        [node, rand, jnp.zeros((N, DP - D_OUT), jnp.float32)], axis=1)
    dst = edge_index[0]
    src = edge_index[1]
    zeros_blk = jnp.zeros((ZBLK, DP), jnp.float32)
    partial_acc = _sc_gather_scatter_add(node_pad, src, dst, zeros_blk)
    return _combine(partial_acc, partial_acc)


# speedup vs baseline: 5.1338x; 5.1338x over previous
import functools

import jax
import jax.numpy as jnp
from jax import lax
from jax.experimental import pallas as pl
from jax.experimental.pallas import tpu as pltpu
from jax.experimental.pallas import tpu_sc as plsc

N = 10000
NPAD = 10240
M = 320000
D_IN = 128
D_OUT = 129
DP = 144
NC = 2
NS = 16
NW = NC * NS
EDGES_PER_TILE = M // NW
CHUNK = 80
N_CHUNKS = EDGES_PER_TILE // CHUNK
ROWS_PER_TILE = NPAD // NS
ZBLK = 128

_MESH = plsc.VectorSubcoreMesh(core_axis_name="c", subcore_axis_name="s")


@functools.partial(
    pl.kernel,
    out_type=(jax.ShapeDtypeStruct((NPAD, DP), jnp.float32),
              jax.ShapeDtypeStruct((NPAD, DP), jnp.float32)),
    mesh=_MESH,
    scratch_types=[
        pltpu.VMEM((CHUNK,), jnp.int32),
        pltpu.VMEM((CHUNK,), jnp.int32),
        pltpu.VMEM((CHUNK, DP), jnp.float32),
        pltpu.VMEM_SHARED((NPAD, DP), jnp.float32),
        pltpu.SemaphoreType.DMA,
    ],
    compiler_params=pltpu.CompilerParams(use_tc_tiling_on_sc=False),
)
def _sc_gather_scatter_add(node_hbm, src_hbm, dst_hbm, zeros_hbm,
                           out0_hbm, out1_hbm,
                           src_v, dst_v, rows_v, acc, sem):
    c = lax.axis_index("c")
    s = lax.axis_index("s")
    wid = s * NC + c
    row0 = pl.multiple_of(s * ROWS_PER_TILE, ZBLK)

    @pl.when(c == 0)
    def _():
        pltpu.sync_copy(node_hbm.at[pl.ds(row0, ROWS_PER_TILE)],
                        acc.at[pl.ds(row0, ROWS_PER_TILE)])

    @pl.when(c != 0)
    def _():
        for j in range(ROWS_PER_TILE // ZBLK):
            pltpu.sync_copy(zeros_hbm, acc.at[pl.ds(row0 + j * ZBLK, ZBLK)])

    plsc.subcore_barrier()

    e0 = wid * EDGES_PER_TILE

    def body(i, carry):
        base = e0 + i * CHUNK
        pltpu.sync_copy(src_hbm.at[pl.ds(base, CHUNK)], src_v)
        pltpu.sync_copy(dst_hbm.at[pl.ds(base, CHUNK)], dst_v)
        pltpu.async_copy(node_hbm.at[src_v], rows_v, sem).wait()
        pltpu.sync_copy(rows_v, acc.at[dst_v], add=True)
        return carry

    lax.fori_loop(0, N_CHUNKS, body, 0)

    plsc.subcore_barrier()

    @pl.when(c == 0)
    def _():
        pltpu.sync_copy(acc.at[pl.ds(row0, ROWS_PER_TILE)],
                        out0_hbm.at[pl.ds(row0, ROWS_PER_TILE)])

    @pl.when(c != 0)
    def _():
        pltpu.sync_copy(acc.at[pl.ds(row0, ROWS_PER_TILE)],
                        out1_hbm.at[pl.ds(row0, ROWS_PER_TILE)])


def _combine_body(a_ref, b_ref, o_ref):
    o_ref[...] = (a_ref[...] + b_ref[...])[:, :D_OUT]


_combine = pl.pallas_call(
    _combine_body,
    grid=(5,),
    in_specs=[
        pl.BlockSpec((N // 5, DP), lambda i: (i, 0)),
        pl.BlockSpec((N // 5, DP), lambda i: (i, 0)),
    ],
    out_specs=pl.BlockSpec((N // 5, D_OUT), lambda i: (i, 0)),
    out_shape=jax.ShapeDtypeStruct((N, D_OUT), jnp.float32),
)


def kernel(node, edge_index, eps_k):
    del eps_k
    rkey = jax.random.fold_in(jax.random.key(0), 42)
    rand = jax.random.uniform(
        rkey, (N, 1), minval=0.0, maxval=100.0, dtype=jnp.float32) / 100.0
    node_pad = jnp.zeros((NPAD, DP), jnp.float32)
    node_pad = node_pad.at[:N, :D_IN].set(node)
    node_pad = node_pad.at[:N, D_IN:D_OUT].set(rand)
    dst = edge_index[0]
    src = edge_index[1]
    zeros_blk = jnp.zeros((ZBLK, DP), jnp.float32)
    p0, p1 = _sc_gather_scatter_add(node_pad, src, dst, zeros_blk)
    return _combine(p0, p1)


# speedup vs baseline: 8.1626x; 1.5900x over previous
import functools

import jax
import jax.numpy as jnp
from jax import lax
from jax.experimental import pallas as pl
from jax.experimental.pallas import tpu as pltpu
from jax.experimental.pallas import tpu_sc as plsc

N = 10000
NPAD = 10240
M = 320000
D_IN = 128
D_OUT = 129
DP = 144
NC = 2
NS = 16
NW = NC * NS
EDGES_PER_TILE = M // NW
CHUNK = 40
N_CHUNKS = EDGES_PER_TILE // CHUNK
NBUF = 5
N_GROUPS = N_CHUNKS // NBUF
ROWS_PER_TILE = NPAD // NS
ZBLK = 128

_MESH = plsc.VectorSubcoreMesh(core_axis_name="c", subcore_axis_name="s")


@functools.partial(
    pl.kernel,
    out_type=(jax.ShapeDtypeStruct((NPAD, DP), jnp.float32),
              jax.ShapeDtypeStruct((NPAD, DP), jnp.float32)),
    mesh=_MESH,
    scratch_types=[
        pltpu.VMEM((NBUF, CHUNK), jnp.int32),
        pltpu.VMEM((NBUF, CHUNK), jnp.int32),
        pltpu.VMEM((NBUF, CHUNK, DP), jnp.float32),
        pltpu.VMEM_SHARED((NPAD, DP), jnp.float32),
        pltpu.SemaphoreType.DMA((NBUF,)),
        pltpu.SemaphoreType.DMA((NBUF,)),
        pltpu.SemaphoreType.DMA((NBUF,)),
    ],
    compiler_params=pltpu.CompilerParams(use_tc_tiling_on_sc=False),
)
def _sc_gather_scatter_add(node_hbm, src_hbm, dst_hbm, zeros_hbm,
                           out0_hbm, out1_hbm,
                           src_v, dst_v, rows_v, acc, sem_i, sem_g, sem_a):
    c = lax.axis_index("c")
    s = lax.axis_index("s")
    wid = s * NC + c
    row0 = pl.multiple_of(s * ROWS_PER_TILE, ZBLK)

    @pl.when(c == 0)
    def _():
        pltpu.sync_copy(node_hbm.at[pl.ds(row0, ROWS_PER_TILE)],
                        acc.at[pl.ds(row0, ROWS_PER_TILE)])

    @pl.when(c != 0)
    def _():
        for j in range(ROWS_PER_TILE // ZBLK):
            pltpu.sync_copy(zeros_hbm, acc.at[pl.ds(row0 + j * ZBLK, ZBLK)])

    plsc.subcore_barrier()

    e0 = wid * EDGES_PER_TILE

    def _idx_copies(chunk, b):
        base = e0 + chunk * CHUNK
        return (
            pltpu.make_async_copy(src_hbm.at[pl.ds(base, CHUNK)],
                                  src_v.at[b], sem_i.at[b]),
            pltpu.make_async_copy(dst_hbm.at[pl.ds(base, CHUNK)],
                                  dst_v.at[b], sem_i.at[b]),
        )

    for b in range(NBUF):
        for d in _idx_copies(b, b):
            d.start()

    def group(g, carry):
        gathers = []
        for b in range(NBUF):
            for d in _idx_copies(g * NBUF + b, b):
                d.wait()
            d = pltpu.make_async_copy(node_hbm.at[src_v.at[b]],
                                      rows_v.at[b], sem_g.at[b])
            d.start()
            gathers.append(d)
        scatters = []
        for b in range(NBUF):
            gathers[b].wait()
            d = pltpu.async_copy(rows_v.at[b], acc.at[dst_v.at[b]],
                                 sem_a.at[b], add=True)
            scatters.append(d)
        for b in range(NBUF):
            scatters[b].wait()

            @pl.when(g < N_GROUPS - 1)
            def _():
                for d in _idx_copies((g + 1) * NBUF + b, b):
                    d.start()

        return carry

    lax.fori_loop(0, N_GROUPS, group, 0)

    plsc.subcore_barrier()

    @pl.when(c == 0)
    def _():
        pltpu.sync_copy(acc.at[pl.ds(row0, ROWS_PER_TILE)],
                        out0_hbm.at[pl.ds(row0, ROWS_PER_TILE)])

    @pl.when(c != 0)
    def _():
        pltpu.sync_copy(acc.at[pl.ds(row0, ROWS_PER_TILE)],
                        out1_hbm.at[pl.ds(row0, ROWS_PER_TILE)])


def _combine_body(a_ref, b_ref, o_ref):
    o_ref[...] = (a_ref[...] + b_ref[...])[:, :D_OUT]


_combine = pl.pallas_call(
    _combine_body,
    grid=(5,),
    in_specs=[
        pl.BlockSpec((N // 5, DP), lambda i: (i, 0)),
        pl.BlockSpec((N // 5, DP), lambda i: (i, 0)),
    ],
    out_specs=pl.BlockSpec((N // 5, D_OUT), lambda i: (i, 0)),
    out_shape=jax.ShapeDtypeStruct((N, D_OUT), jnp.float32),
)


def kernel(node, edge_index, eps_k):
    del eps_k
    rkey = jax.random.fold_in(jax.random.key(0), 42)
    rand = jax.random.uniform(
        rkey, (N, 1), minval=0.0, maxval=100.0, dtype=jnp.float32) / 100.0
    node_pad = jnp.zeros((NPAD, DP), jnp.float32)
    node_pad = node_pad.at[:N, :D_IN].set(node)
    node_pad = node_pad.at[:N, D_IN:D_OUT].set(rand)
    dst = edge_index[0]
    src = edge_index[1]
    zeros_blk = jnp.zeros((ZBLK, DP), jnp.float32)
    p0, p1 = _sc_gather_scatter_add(node_pad, src, dst, zeros_blk)
    return _combine(p0, p1)
